# baseline (speedup 1.0000x reference)
import functools

import numpy as np
import jax
import jax.numpy as jnp
from jax import lax
from jax.experimental import pallas as pl
from jax.experimental.pallas import tpu as pltpu
import jax.experimental.pallas.tpu_sc as plsc

_B, _L, _D = 4096, 50, 64
_NC, _NS = 2, 16
_NW = _NC * _NS
_BPW = _B // _NW
_PPW = _BPW * _L
_SLEN = 2 * _L
_NSTREAM = _BPW // 2
_NBUF = 4
_SP = 40
_NV4 = _D // 16


def _make_body(nf):
    def body(*refs):
        (idxs, dst, zeros, tabs, out, idxv, dstv, rows, outbuf,
         accs, gsem, ssem, zsem, osem) = (
            refs[0:nf], refs[nf], refs[nf + 1], refs[nf + 2:2 * nf + 2],
            refs[2 * nf + 2], refs[2 * nf + 3], refs[2 * nf + 4],
            refs[2 * nf + 5], refs[2 * nf + 6],
            refs[2 * nf + 7:3 * nf + 7],
            refs[3 * nf + 7], refs[3 * nf + 8], refs[3 * nf + 9],
            refs[3 * nf + 10])
        sid = lax.axis_index("s")
        wid = sid * _NC + lax.axis_index("c")
        base = wid * _BPW
        my0 = sid * _BPW

        for f in range(nf):
            pltpu.sync_copy(idxs[f].at[wid], idxv.at[f])
        pltpu.sync_copy(dst.at[sid], dstv)
        for f in range(nf):
            pltpu.async_copy(zeros.at[pl.ds(0, 2 * _SP)],
                             accs[f].at[pl.ds(my0, 2 * _SP)], zsem)
        for f in range(nf):
            pltpu.make_async_copy(zeros.at[pl.ds(0, 2 * _SP)],
                                  accs[f].at[pl.ds(my0, 2 * _SP)],
                                  zsem).wait()

        def gather(f, s, slot):
            pltpu.async_copy(tabs[f].at[idxv.at[f, s]], rows.at[slot],
                             gsem.at[slot])

        def wait_gather(f, slot):
            pltpu.make_async_copy(tabs[f].at[idxv.at[f, 0]], rows.at[slot],
                                  gsem.at[slot]).wait()

        def scatter(f, s, slot):
            pltpu.async_copy(rows.at[slot], accs[f].at[dstv.at[s]],
                             ssem.at[slot], add=True)

        def wait_scatter(f, slot):
            pltpu.make_async_copy(rows.at[slot], accs[f].at[dstv.at[0]],
                                  ssem.at[slot]).wait()

        def valu(f, s, slot):
            for h in range(2):
                def red(l, acc):
                    return tuple(
                        acc[j] + rows[slot, h * _L + l, pl.ds(16 * j, 16)]
                        for j in range(_NV4))

                acc0 = tuple(rows[slot, h * _L, pl.ds(16 * j, 16)]
                             for j in range(_NV4))
                accv = lax.fori_loop(1, _L, red, acc0, unroll=7)
                for j in range(_NV4):
                    outbuf[f, 2 * s + h, pl.ds(16 * j, 16)] = accv[j]

        for f in range(nf):
            for b in range(_NBUF):
                gather(f, b, b)
            wait_gather(f, 0)
            scatter(f, 0, 0)

            def stepA(s, carry):
                p = s % _NBUF
                p1 = (s - 1) % _NBUF
                wait_gather(f, p)
                scatter(f, s, p)
                wait_scatter(f, p1)
                gather(f, s - 1 + _NBUF, p1)
                return carry

            lax.fori_loop(1, _SP, stepA, 0, unroll=2)

            for s in range(_SP, _SP + _NBUF):
                p = s % _NBUF
                if s == _SP + _NBUF - 1:
                    wait_scatter(f, p)
                    gather(f, s, p)
                wait_gather(f, p)
                valu(f, s, p)
                ns = s + _NBUF
                if ns < _NSTREAM:
                    gather(f, ns, p)

            def stepB(s, carry):
                p = s % _NBUF
                wait_gather(f, p)
                valu(f, s, p)
                gather(f, s + _NBUF, p)
                return carry

            lax.fori_loop(_SP + _NBUF, _NSTREAM - _NBUF, stepB, 0)
            for s in range(max(_SP + _NBUF, _NSTREAM - _NBUF), _NSTREAM):
                p = s % _NBUF
                wait_gather(f, p)
                valu(f, s, p)

            pltpu.async_copy(
                accs[f].at[pl.ds(my0, 2 * _SP)],
                out.at[pl.ds(base, 2 * _SP), pl.ds(f * _D, _D)], osem)
            pltpu.async_copy(
                outbuf.at[f, pl.ds(2 * _SP, _BPW - 2 * _SP)],
                out.at[pl.ds(base + 2 * _SP, _BPW - 2 * _SP),
                       pl.ds(f * _D, _D)], osem)

        for f in range(nf):
            pltpu.make_async_copy(
                accs[f].at[pl.ds(my0, 2 * _SP)],
                out.at[pl.ds(base, 2 * _SP), pl.ds(f * _D, _D)], osem).wait()
            pltpu.make_async_copy(
                outbuf.at[f, pl.ds(2 * _SP, _BPW - 2 * _SP)],
                out.at[pl.ds(base + 2 * _SP, _BPW - 2 * _SP),
                       pl.ds(f * _D, _D)], osem).wait()

    return body


_DST = (np.arange(_NS, dtype=np.int32)[:, None, None] * _BPW
        + (np.arange(_PPW, dtype=np.int32) // _L
           ).reshape(_NSTREAM, _SLEN)[None]).astype(np.int32)
_ZEROS = np.zeros((_BPW, _D), np.float32)

_NREP = 8
_REP_OFF = ((np.arange(_NSTREAM, dtype=np.int32) % _NREP) * 1000)[None, :, None]


def _make_kernel(nf):
    acc_t = pltpu.VMEM_SHARED((_NS * _BPW, _D), jnp.float32)
    return pl.kernel(
        _make_body(nf),
        out_type=jax.ShapeDtypeStruct((_B, nf * _D), jnp.float32),
        mesh=plsc.VectorSubcoreMesh(core_axis_name="c", subcore_axis_name="s"),
        compiler_params=pltpu.CompilerParams(use_tc_tiling_on_sc=False),
        scratch_types=[
            pltpu.VMEM((nf, _NSTREAM, _SLEN), jnp.int32),
            pltpu.VMEM((_NSTREAM, _SLEN), jnp.int32),
            pltpu.VMEM((_NBUF, _SLEN, _D), jnp.float32),
            pltpu.VMEM((nf, _BPW, _D), jnp.float32),
        ] + [acc_t] * nf + [
            pltpu.SemaphoreType.DMA((_NBUF,)),
            pltpu.SemaphoreType.DMA((_NBUF,)),
            pltpu.SemaphoreType.DMA,
            pltpu.SemaphoreType.DMA,
        ],
    )


@jax.jit
def kernel(vgids, vsids, vcids, vgprices,
           goods_table, shop_table, cate_table, price_table):
    shape3 = (_NW, _NSTREAM, _SLEN)
    gidx = vgids.astype(jnp.int32).reshape(shape3)
    sidx = vsids.astype(jnp.int32).reshape(shape3)
    rep_off = jnp.asarray(_REP_OFF)
    cidx = vcids.astype(jnp.int32).reshape(shape3) + rep_off
    pidx = vgprices.astype(jnp.int32).reshape(shape3) + rep_off
    dst = jnp.asarray(_DST)
    zeros = jnp.asarray(_ZEROS)
    cate8 = jnp.tile(cate_table, (_NREP, 1))
    price8 = jnp.tile(price_table, (_NREP, 1))

    out_cp = _make_kernel(2)(cidx, pidx, dst, zeros, cate8, price8)
    out_g = _make_kernel(1)(gidx, dst, zeros, goods_table)
    out_s = _make_kernel(1)(sidx, dst, zeros, shop_table)
    return jnp.concatenate([out_g, out_s, out_cp], axis=1)


# speedup vs baseline: 1.0106x; 1.0106x over previous
import functools

import numpy as np
import jax
import jax.numpy as jnp
from jax import lax
from jax.experimental import pallas as pl
from jax.experimental.pallas import tpu as pltpu
import jax.experimental.pallas.tpu_sc as plsc

_B, _L, _D = 4096, 50, 64
_NC, _NS = 2, 16
_NW = _NC * _NS
_BPW = _B // _NW
_PPW = _BPW * _L
_CHUNK = 128
_NCHUNK = _PPW // _CHUNK
_CB = 5
_NSTEP = _NCHUNK // _CB
_NBUF = 2


def _body(idx0, idx1, dst, zeros, tab0, tab1, out,
          idxv, dstv, rows, acc0, acc1, gsem, ssem, zsem):
    sid = lax.axis_index("s")
    wid = sid * _NC + lax.axis_index("c")
    base = wid * _BPW
    accs = (acc0, acc1)
    tables = (tab0, tab1)

    for f, idx_hbm in enumerate((idx0, idx1)):
        pltpu.sync_copy(idx_hbm.at[wid], idxv.at[f])
    pltpu.sync_copy(dst.at[sid], dstv)
    my = pl.ds(sid * _BPW, _BPW)
    for f in range(2):
        pltpu.async_copy(zeros, accs[f].at[my], zsem)
    for f in range(2):
        pltpu.make_async_copy(zeros, accs[f].at[my], zsem).wait()

    pending = [False] * _NBUF

    def gather(f, j, slot):
        pltpu.async_copy(tables[f].at[idxv.at[f, j]], rows.at[slot],
                         gsem.at[slot])

    def wait_gather(f, slot):
        pltpu.make_async_copy(tables[f].at[idxv.at[f, 0]], rows.at[slot],
                              gsem.at[slot]).wait()

    def scatter(f, j, slot):
        pltpu.async_copy(rows.at[slot], accs[f].at[dstv.at[j]],
                         ssem.at[slot], add=True)

    def wait_scatter(f, slot):
        pltpu.make_async_copy(rows.at[slot], accs[f].at[dstv.at[0]],
                              ssem.at[slot]).wait()

    for f in range(2):
        for b in range(_NBUF):
            if pending[b]:
                wait_scatter(f - 1, b)
                pending[b] = False
            gather(f, b, b)
        wait_gather(f, 0)
        scatter(f, 0, 0)

        def step(j, carry):
            p = j % _NBUF
            p1 = (j - 1) % _NBUF
            wait_gather(f, p)
            scatter(f, j, p)
            wait_scatter(f, p1)
            gather(f, j - 1 + _NBUF, p1)
            return carry

        lax.fori_loop(1, _NSTEP - _NBUF + 1, step, 0, unroll=2)

        for j in range(_NSTEP - _NBUF + 1, _NSTEP):
            p = j % _NBUF
            wait_gather(f, p)
            scatter(f, j, p)
        for j in range(_NSTEP - _NBUF, _NSTEP):
            pending[j % _NBUF] = True

    for b in range(_NBUF):
        if pending[b]:
            wait_scatter(1, b)
            pending[b] = False
    for f in range(2):
        pltpu.sync_copy(accs[f].at[my],
                        out.at[pl.ds(base, _BPW), pl.ds(f * _D, _D)])


_LOCAL = (np.arange(_PPW, dtype=np.int32) // _L).reshape(_NSTEP, _CB * _CHUNK)
_DST = (np.arange(_NS, dtype=np.int32)[:, None, None] * _BPW
        + _LOCAL[None]).astype(np.int32)
_ZEROS = np.zeros((_BPW, _D), np.float32)

_NREP = 8
_REP_OFF = ((np.arange(_NSTEP, dtype=np.int32) % _NREP) * 1000)[None, :, None]


def _make_pair_kernel():
    acc_t = pltpu.VMEM_SHARED((_NS * _BPW, _D), jnp.float32)
    return pl.kernel(
        _body,
        out_type=jax.ShapeDtypeStruct((_B, 2 * _D), jnp.float32),
        mesh=plsc.VectorSubcoreMesh(core_axis_name="c", subcore_axis_name="s"),
        compiler_params=pltpu.CompilerParams(use_tc_tiling_on_sc=False),
        scratch_types=[
            pltpu.VMEM((2, _NSTEP, _CB * _CHUNK), jnp.int32),
            pltpu.VMEM((_NSTEP, _CB * _CHUNK), jnp.int32),
            pltpu.VMEM((_NBUF, _CB * _CHUNK, _D), jnp.float32),
            acc_t, acc_t,
            pltpu.SemaphoreType.DMA((_NBUF,)),
            pltpu.SemaphoreType.DMA((_NBUF,)),
            pltpu.SemaphoreType.DMA,
        ],
    )


@jax.jit
def kernel(vgids, vsids, vcids, vgprices,
           goods_table, shop_table, cate_table, price_table):
    shape3 = (_NW, _NSTEP, _CB * _CHUNK)
    gidx = vgids.astype(jnp.int32).reshape(shape3)
    sidx = vsids.astype(jnp.int32).reshape(shape3)
    cidx = vcids.astype(jnp.int32).reshape(shape3)
    pidx = vgprices.astype(jnp.int32).reshape(shape3)
    dst = jnp.asarray(_DST)
    zeros = jnp.asarray(_ZEROS)

    rep_off = jnp.asarray(_REP_OFF)
    cidx = cidx + rep_off
    pidx = pidx + rep_off
    cate8 = jnp.tile(cate_table, (_NREP, 1))
    price8 = jnp.tile(price_table, (_NREP, 1))

    run = _make_pair_kernel()
    out_cp = run(cidx, pidx, dst, zeros, cate8, price8)
    out_gs = run(gidx, sidx, dst, zeros, goods_table, shop_table)
    return jnp.concatenate([out_gs, out_cp], axis=1)


# speedup vs baseline: 1.0910x; 1.0795x over previous
import functools

import numpy as np
import jax
import jax.numpy as jnp
from jax import lax
from jax.experimental import pallas as pl
from jax.experimental.pallas import tpu as pltpu
import jax.experimental.pallas.tpu_sc as plsc

_B, _L, _D = 4096, 50, 64
_NC, _NS = 2, 16
_NW = _NC * _NS
_BPW = _B // _NW
_PPW = _BPW * _L
_SLEN = 2 * _L
_NSTREAM = _BPW // 2
_GS = 5
_GV = 3
_NBUF = _GS + _GV
_NGRP = _NSTREAM // _NBUF
_NSC = _GS * _NGRP
_NV4 = _D // 16


def _make_body(nf):
    def body(*refs):
        idxs = refs[0:nf]
        dst, zeros = refs[nf], refs[nf + 1]
        tabs = refs[nf + 2:2 * nf + 2]
        out = refs[2 * nf + 2]
        idxv, dstv, rows, outbuf = refs[2 * nf + 3:2 * nf + 7]
        accs = refs[2 * nf + 7:3 * nf + 7]
        gsem, ssem, zsem, osem = refs[3 * nf + 7:3 * nf + 11]
        sid = lax.axis_index("s")
        wid = sid * _NC + lax.axis_index("c")
        base = wid * _BPW
        my0 = sid * _BPW

        for f in range(nf):
            pltpu.sync_copy(idxs[f].at[wid], idxv.at[f])
        pltpu.sync_copy(dst.at[sid], dstv)
        for f in range(nf):
            pltpu.async_copy(zeros.at[pl.ds(0, 2 * _NSC)],
                             accs[f].at[pl.ds(my0, 2 * _NSC)], zsem)
        for f in range(nf):
            pltpu.make_async_copy(zeros.at[pl.ds(0, 2 * _NSC)],
                                  accs[f].at[pl.ds(my0, 2 * _NSC)],
                                  zsem).wait()

        def stream_of(g, k):
            return _GS * g + k if k < _GS else _NSC + _GV * g + (k - _GS)

        def gather(f, g, k):
            pltpu.async_copy(tabs[f].at[idxv.at[f, stream_of(g, k)]],
                             rows.at[k], gsem.at[k])

        def wait_gather(f, k):
            pltpu.make_async_copy(tabs[f].at[idxv.at[f, 0]], rows.at[k],
                                  gsem.at[k]).wait()

        def scatter(f, g, k):
            pltpu.async_copy(rows.at[k], accs[f].at[dstv.at[_GS * g + k]],
                             ssem.at[k], add=True)

        def wait_scatter(f, k):
            pltpu.make_async_copy(rows.at[k], accs[f].at[dstv.at[0]],
                                  ssem.at[k]).wait()

        def valu(f, g, k):
            s = _NSC + _GV * g + (k - _GS)
            for h in range(2):
                def red(l, acc):
                    return tuple(
                        acc[j] + rows[k, h * _L + l, pl.ds(16 * j, 16)]
                        for j in range(_NV4))

                acc0 = tuple(rows[k, h * _L, pl.ds(16 * j, 16)]
                             for j in range(_NV4))
                accv = lax.fori_loop(1, _L, red, acc0, unroll=7)
                for j in range(_NV4):
                    outbuf[f, 2 * s + h, pl.ds(16 * j, 16)] = accv[j]

        def group(f, g, refill):
            for k in range(_GS):
                wait_gather(f, k)
                scatter(f, g, k)
            for k in range(_GS, _NBUF):
                wait_gather(f, k)
                valu(f, g, k)
                if refill:
                    gather(f, g + 1, k)
            for k in range(_GS):
                wait_scatter(f, k)
                if refill:
                    gather(f, g + 1, k)

        for f in range(nf):
            for k in range(_NBUF):
                gather(f, 0, k)
            lax.fori_loop(0, _NGRP - 1,
                          lambda g, c: (group(f, g, True), c)[1], 0)
            group(f, _NGRP - 1, False)

            pltpu.async_copy(
                accs[f].at[pl.ds(my0, 2 * _NSC)],
                out.at[pl.ds(base, 2 * _NSC), pl.ds(f * _D, _D)], osem)
            pltpu.async_copy(
                outbuf.at[f, pl.ds(2 * _NSC, _BPW - 2 * _NSC)],
                out.at[pl.ds(base + 2 * _NSC, _BPW - 2 * _NSC),
                       pl.ds(f * _D, _D)], osem)

        for f in range(nf):
            pltpu.make_async_copy(
                accs[f].at[pl.ds(my0, 2 * _NSC)],
                out.at[pl.ds(base, 2 * _NSC), pl.ds(f * _D, _D)], osem).wait()
            pltpu.make_async_copy(
                outbuf.at[f, pl.ds(2 * _NSC, _BPW - 2 * _NSC)],
                out.at[pl.ds(base + 2 * _NSC, _BPW - 2 * _NSC),
                       pl.ds(f * _D, _D)], osem).wait()

    return body


_DST = (np.arange(_NS, dtype=np.int32)[:, None, None] * _BPW
        + (np.arange(_NSC * _SLEN, dtype=np.int32) // _L
           ).reshape(_NSC, _SLEN)[None]).astype(np.int32)
_ZEROS = np.zeros((_BPW, _D), np.float32)

_NREP = 8
_REP_OFF = ((np.arange(_NSTREAM, dtype=np.int32) % _NREP) * 1000)[None, :, None]


def _make_kernel(nf):
    acc_t = pltpu.VMEM_SHARED((_NS * _BPW, _D), jnp.float32)
    return pl.kernel(
        _make_body(nf),
        out_type=jax.ShapeDtypeStruct((_B, nf * _D), jnp.float32),
        mesh=plsc.VectorSubcoreMesh(core_axis_name="c", subcore_axis_name="s"),
        compiler_params=pltpu.CompilerParams(use_tc_tiling_on_sc=False),
        scratch_types=[
            pltpu.VMEM((nf, _NSTREAM, _SLEN), jnp.int32),
            pltpu.VMEM((_NSC, _SLEN), jnp.int32),
            pltpu.VMEM((_NBUF, _SLEN, _D), jnp.float32),
            pltpu.VMEM((nf, _BPW, _D), jnp.float32),
        ] + [acc_t] * nf + [
            pltpu.SemaphoreType.DMA((_NBUF,)),
            pltpu.SemaphoreType.DMA((_NBUF,)),
            pltpu.SemaphoreType.DMA,
            pltpu.SemaphoreType.DMA,
        ],
    )


@jax.jit
def kernel(vgids, vsids, vcids, vgprices,
           goods_table, shop_table, cate_table, price_table):
    shape3 = (_NW, _NSTREAM, _SLEN)
    gidx = vgids.astype(jnp.int32).reshape(shape3)
    sidx = vsids.astype(jnp.int32).reshape(shape3)
    rep_off = jnp.asarray(_REP_OFF)
    cidx = vcids.astype(jnp.int32).reshape(shape3) + rep_off
    pidx = vgprices.astype(jnp.int32).reshape(shape3) + rep_off
    dst = jnp.asarray(_DST)
    zeros = jnp.asarray(_ZEROS)
    cate8 = jnp.tile(cate_table, (_NREP, 1))
    price8 = jnp.tile(price_table, (_NREP, 1))

    run = _make_kernel(2)
    out_cp = run(cidx, pidx, dst, zeros, cate8, price8)
    out_gs = run(gidx, sidx, dst, zeros, goods_table, shop_table)
    return jnp.concatenate([out_gs, out_cp], axis=1)
